# Initial kernel scaffold; baseline (speedup 1.0000x reference)
import jax
import jax.numpy as jnp
from jax.experimental import pallas as pl


def kernel(x, edge_index, edge_attr, xemb1, xemb2, eemb1, eemb2, eemb3, W1, b1, W2, b2, bn_w, bn_b):
    raise NotImplementedError("write your pallas kernel here")


# speedup vs baseline: 2.1245x; 2.1245x over previous
import functools

import numpy as np
import jax
import jax.numpy as jnp
from jax import lax
from jax.experimental import pallas as pl
from jax.experimental.pallas import tpu as pltpu
from jax.experimental.pallas import tpu_sc as plsc

NC = 2
NS = 16
NW = NC * NS
CHUNK = 128


@functools.lru_cache(maxsize=None)
def _gather_acc_builder(acc_rows, rw, d):
    mesh = plsc.VectorSubcoreMesh(core_axis_name="c", subcore_axis_name="s")

    @functools.partial(
        pl.kernel,
        out_type=jax.ShapeDtypeStruct((NW * rw, d), jnp.float32),
        mesh=mesh,
        scratch_types=[
            pltpu.VMEM((2, CHUNK), jnp.int32),
            pltpu.VMEM((CHUNK, d), jnp.float32),
            pltpu.VMEM((acc_rows, d), jnp.float32),
            pltpu.VMEM((16,), jnp.int32),
            pltpu.SemaphoreType.DMA,
        ],
    )
    def k(tab_hbm, idx_hbm, meta_hbm, z_hbm, out_hbm, idxv, buf, acc, mets,
          sem):
        c = lax.axis_index("c")
        s = lax.axis_index("s")
        w = c * NS + s
        pltpu.sync_copy(z_hbm, acc)
        pltpu.sync_copy(meta_hbm.at[w], mets)
        mv = mets[pl.ds(0, 16)]
        base = mv[0]
        nch = mv[1]

        def chunk(i, _):
            pltpu.sync_copy(idx_hbm.at[base + i], idxv)
            pltpu.async_copy(tab_hbm.at[idxv.at[0]], buf, sem).wait()

            def group(g, _2):
                ldvec = idxv[1, pl.ds(g * 16, 16)]
                for t in range(16):
                    r = ldvec[t]

                    def col(j, _3, r=r, t=t):
                        plsc.addupdate(acc.at[r, pl.ds(j * 16, 16)],
                                       buf[g * 16 + t, pl.ds(j * 16, 16)])
                        return 0

                    lax.fori_loop(0, d // 16, col, 0, unroll=True)
                return 0

            lax.fori_loop(0, CHUNK // 16, group, 0)
            return 0

        lax.fori_loop(0, nch, chunk, 0)
        pltpu.sync_copy(acc.at[pl.ds(0, rw)], out_hbm.at[pl.ds(w * rw, rw)])

    return k


def _gather_acc_sc(table, idx_r, meta, zeros_hbm, acc_rows, rw, d):
    return _gather_acc_builder(acc_rows, rw, d)(table, idx_r, meta, zeros_hbm)


def _h0_sc(xf_tab, xfi_r, sub_offs, rw, d):
    mesh = plsc.VectorSubcoreMesh(core_axis_name="c", subcore_axis_name="s")
    n_sub = len(sub_offs)

    @functools.partial(
        pl.kernel,
        out_type=jax.ShapeDtypeStruct((NW * rw, d), jnp.float32),
        mesh=mesh,
        scratch_types=[
            pltpu.VMEM((2, CHUNK), jnp.int32),
            pltpu.VMEM((CHUNK, d), jnp.float32),
            pltpu.SemaphoreType.DMA,
        ],
    )
    def k(xf_hbm, xfi_hbm, h0_hbm, xfiv, buf, sem):
        c = lax.axis_index("c")
        s = lax.axis_index("s")
        w = c * NS + s
        for j in range(n_sub):
            pltpu.sync_copy(xfi_hbm.at[w * n_sub + j], xfiv)
            pltpu.async_copy(xf_hbm.at[xfiv.at[0]], buf, sem).wait()
            pltpu.sync_copy(buf, h0_hbm.at[pl.ds(w * rw + sub_offs[j], CHUNK)])

    return k(xf_tab, xfi_r)


def _mlp_tc(aggrh, h, cnts, ecat, selfee, w1, b1p, w2, b2p, nb, br):
    n, d = h.shape
    d2 = w1.shape[1]

    def body(ah_ref, h_ref, c_ref, e_ref, se_ref, w1_ref, b1_ref, w2_ref,
             b2_ref, h2_ref, st_ref, sacc):
        i = pl.program_id(0)
        a = (ah_ref[...] + h_ref[...] + se_ref[0:1, :]
             + jnp.dot(c_ref[...], e_ref[...], preferred_element_type=jnp.float32, precision=lax.Precision.HIGHEST))
        hid = jnp.maximum(
            jnp.dot(a, w1_ref[...], preferred_element_type=jnp.float32, precision=lax.Precision.DEFAULT)
            + b1_ref[0:1, :], 0.0)
        h2 = (jnp.dot(hid, w2_ref[...], preferred_element_type=jnp.float32, precision=lax.Precision.DEFAULT)
              + b2_ref[0:1, :])
        h2_ref[...] = h2

        @pl.when(i == 0)
        def _():
            sacc[...] = jnp.zeros_like(sacc)

        sacc[0:1, :] = sacc[0:1, :] + jnp.sum(h2, axis=0, keepdims=True)
        sacc[1:2, :] = sacc[1:2, :] + jnp.sum(h2 * h2, axis=0, keepdims=True)

        @pl.when(i == nb - 1)
        def _():
            st_ref[...] = sacc[...]

    return pl.pallas_call(
        body,
        grid=(nb,),
        in_specs=[
            pl.BlockSpec((br, d), lambda i: (i, 0)),
            pl.BlockSpec((br, d), lambda i: (i, 0)),
            pl.BlockSpec((br, d), lambda i: (i, 0)),
            pl.BlockSpec((d, d), lambda i: (0, 0)),
            pl.BlockSpec((8, d), lambda i: (0, 0)),
            pl.BlockSpec((d, d2), lambda i: (0, 0)),
            pl.BlockSpec((8, d2), lambda i: (0, 0)),
            pl.BlockSpec((d2, d), lambda i: (0, 0)),
            pl.BlockSpec((8, d), lambda i: (0, 0)),
        ],
        out_specs=[
            pl.BlockSpec((br, d), lambda i: (i, 0)),
            pl.BlockSpec((8, d), lambda i: (0, 0)),
        ],
        out_shape=[
            jax.ShapeDtypeStruct((n, d), jnp.float32),
            jax.ShapeDtypeStruct((8, d), jnp.float32),
        ],
        scratch_shapes=[pltpu.VMEM((8, d), jnp.float32)],
    )(aggrh, h, cnts, ecat, selfee, w1, b1p, w2, b2p)


def _bn_tc(h2, stats, gb, n_total, apply_elu, nb, br):
    n, d = h2.shape

    def body(h2_ref, st_ref, gb_ref, o_ref):
        mean = st_ref[0:1, :] / n_total
        var = st_ref[1:2, :] / n_total - mean * mean
        inv = lax.rsqrt(var + 1e-5)
        y = (h2_ref[...] - mean) * (inv * gb_ref[0:1, :]) + gb_ref[1:2, :]
        if apply_elu:
            y = jnp.where(y > 0, y, jnp.exp(jnp.minimum(y, 0.0)) - 1.0)
        o_ref[...] = y

    return pl.pallas_call(
        body,
        grid=(nb,),
        in_specs=[
            pl.BlockSpec((br, d), lambda i: (i, 0)),
            pl.BlockSpec((8, d), lambda i: (0, 0)),
            pl.BlockSpec((8, d), lambda i: (0, 0)),
        ],
        out_specs=pl.BlockSpec((br, d), lambda i: (i, 0)),
        out_shape=jax.ShapeDtypeStruct((n, d), jnp.float32),
    )(h2, stats, gb)


def kernel(x, edge_index, edge_attr, xemb1, xemb2, eemb1, eemb2, eemb3,
           W1, b1, W2, b2, bn_w, bn_b):
    n = x.shape[0]
    e = edge_attr.shape[0]
    d = xemb1.shape[1]
    n_layers = W1.shape[0]
    na, nch = xemb1.shape[0], xemb2.shape[0]
    nbt, nbd, nbs = eemb1.shape[1], eemb2.shape[1], eemb3.shape[1]

    f32 = jnp.float32
    i32 = jnp.int32

    rw = 8 * (-(-(-(-n // NW)) // 8))
    acc_rows = rw + 8
    ecap = e + NW * CHUNK
    ncht = ecap // CHUNK

    src = edge_index[0].astype(i32)
    dst = edge_index[1].astype(i32)
    perm = jnp.argsort(dst)
    srcs = src[perm]
    dsts = dst[perm]
    wd = dsts // rw
    cnt = jnp.zeros((NW,), i32).at[wd].add(1)
    bstart = jnp.concatenate([jnp.zeros((1,), i32), jnp.cumsum(cnt)[:-1]])
    padcnt = CHUNK * (-(-cnt // CHUNK))
    start = jnp.concatenate([jnp.zeros((1,), i32), jnp.cumsum(padcnt)[:-1]])
    ei = jnp.arange(e, dtype=i32)
    pos = start[wd] + (ei - bstart[wd])
    srcp = jnp.zeros((ecap,), i32).at[pos].set(srcs)
    ldstp = jnp.full((ecap,), rw, i32).at[pos].set(dsts - wd * rw)
    idx_r = jnp.stack(
        [srcp.reshape(ncht, CHUNK), ldstp.reshape(ncht, CHUNK)], axis=1)
    meta = jnp.zeros((NW, 16), i32)
    meta = meta.at[:, 0].set(start // CHUNK).at[:, 1].set(padcnt // CHUNK)

    ea = edge_attr.astype(i32)
    fidx = (jnp.clip(ea[:, 0], 0, nbt - 1)
            + nbt * jnp.clip(ea[:, 1], 0, nbd - 1)
            + nbt * nbd * jnp.clip(ea[:, 2], 0, nbs - 1))
    nv = nbt * nbd * nbs
    nv_pad = 8 * (-(-(nv + 1) // 8))
    fidxp = jnp.full((ecap,), nv, i32).at[pos].set(fidx[perm])
    fidx_r = jnp.stack(
        [fidxp.reshape(ncht, CHUNK), ldstp.reshape(ncht, CHUNK)], axis=1)

    sub_offs = []
    o = 0
    while o + CHUNK < rw:
        sub_offs.append(o)
        o += CHUNK
    sub_offs.append(rw - CHUNK)
    xi = x.astype(i32)
    xfi = jnp.clip(xi[:, 0], 0, na - 1) * nch + jnp.clip(xi[:, 1], 0, nch - 1)
    xfi_pad = jnp.zeros((NW * rw,), i32).at[:n].set(xfi)
    gidx = (jnp.arange(NW, dtype=i32)[:, None, None] * rw
            + jnp.asarray(sub_offs, i32)[None, :, None]
            + jnp.arange(CHUNK, dtype=i32)[None, None, :])
    xfi_r = jnp.zeros((NW * len(sub_offs), 2, CHUNK), i32).at[:, 0, :].set(
        xfi_pad[gidx].reshape(NW * len(sub_offs), CHUNK))

    xf_tab = (xemb1[:, None, :] + xemb2[None, :, :]).reshape(na * nch, d)
    oh_np = np.zeros((nv_pad, d), np.float32)
    v = np.arange(nv)
    oh_np[v, v % nbt] = 1.0
    oh_np[v, nbt + (v // nbt) % nbd] += 1.0
    oh_np[v, nbt + nbd + v // (nbt * nbd)] += 1.0
    oh_tab = jnp.asarray(oh_np)
    ecat = jnp.zeros((n_layers, d, d), f32)
    ecat = ecat.at[:, 0:nbt, :].set(eemb1)
    ecat = ecat.at[:, nbt:nbt + nbd, :].set(eemb2)
    ecat = ecat.at[:, nbt + nbd:nbt + nbd + nbs, :].set(eemb3)
    selfee = eemb1[:, 4, :] + eemb2[:, 0, :] + eemb3[:, 0, :]
    selfee_p = jnp.zeros((n_layers, 8, d), f32).at[:, 0, :].set(selfee)
    b1_p = jnp.zeros((n_layers, 8, 2 * d), f32).at[:, 0, :].set(b1)
    b2_p = jnp.zeros((n_layers, 8, d), f32).at[:, 0, :].set(b2)
    gb = jnp.zeros((n_layers, 8, d), f32)
    gb = gb.at[:, 0, :].set(bn_w).at[:, 1, :].set(bn_b)

    zeros_hbm = jnp.zeros((acc_rows, d), f32)

    h = _h0_sc(xf_tab, xfi_r, sub_offs, rw, d)[:n]
    cnts = _gather_acc_sc(oh_tab, fidx_r, meta, zeros_hbm, acc_rows, rw, d)[:n]

    nb = 10 if (n % 10 == 0 and (n // 10) % 8 == 0) else 1
    br = n // nb

    for l in range(n_layers):
        aggrh = _gather_acc_sc(h, idx_r, meta, zeros_hbm, acc_rows, rw, d)[:n]
        h2, stats = _mlp_tc(aggrh, h, cnts, ecat[l], selfee_p[l],
                            W1[l], b1_p[l], W2[l], b2_p[l], nb, br)
        h = _bn_tc(h2, stats, gb[l], float(n), l < n_layers - 1, nb, br)

    return h
